# baseline (speedup 1.0000x reference)
import jax
import jax.numpy as jnp
from jax.experimental import pallas as pl
from jax.experimental.pallas import tpu as pltpu

_B, _T, _N, _D, _H = 16, 12, 128, 64, 128
_NUM_MAT = 5
_M = _B * _N


def _mm(a, b):
    return jax.lax.dot_general(a, b, (((1,), (0,)), ((), ())),
                               preferred_element_type=jnp.float32)


def _mmr(a, b):
    return jax.lax.dot_general(a, b, (((0,), (0,)), ((), ())),
                               preferred_element_type=jnp.float32)


def _body(x_ref, wi0_ref, wg0s_ref, bg0_ref, wc0s_ref, bc0_ref,
          wi1_ref, wg1s_ref, bg1_ref, wc1s_ref, bc1_ref,
          wfc_ref, bfc_ref, out_ref):
    row = jax.lax.broadcasted_iota(jnp.int32, (_N, _N), 0)
    col = jax.lax.broadcasted_iota(jnp.int32, (_N, _N), 1)
    eye = jnp.where(row == col, 1.0, 0.0).astype(jnp.float32)
    sup = []
    xs = []
    for i in range(_B):
        xt_i = jnp.transpose(x_ref[i], (0, 2, 1)).reshape(_T * _D, _N)
        xs.append(xt_i)
        mu = jnp.sum(xt_i, axis=0, keepdims=True) * (1.0 / (_T * _D))
        xm = xt_i - mu
        cov = _mmr(xm, xm)
        sq = jnp.sum(xm * xm, axis=0, keepdims=True)
        var = jnp.sqrt(jnp.maximum(sq, 1e-12))
        adj = jnp.abs(cov) / (jnp.transpose(var) * var)
        rs = jnp.sum(adj, axis=1, keepdims=True)
        cs = jnp.sum(adj, axis=0, keepdims=True)
        s0t = jnp.transpose(adj) / jnp.transpose(rs)
        s1t = adj / cs
        sup.append(jnp.concatenate(
            [s0t, _mm(s0t, s0t) * 2.0 - eye, s1t,
             _mm(s1t, s1t) * 2.0 - eye], axis=1))

    def diffuse(v):
        prods = [_mm(v[:, i * _N:(i + 1) * _N], sup[i]) for i in range(_B)]
        return tuple(
            jnp.concatenate([p[:, k * _N:(k + 1) * _N] for p in prods],
                            axis=1) for k in range(4))

    wi0 = wi0_ref[...]
    wg0s = wg0s_ref[...]
    bg0 = bg0_ref[...]
    wc0s = wc0s_ref[...]
    bc0 = bc0_ref[...]
    wi1 = wi1_ref[...]
    wg1s = wg1s_ref[...]
    bg1 = bg1_ref[...]
    wc1s = wc1s_ref[...]
    bc1 = bc1_ref[...]

    def gru(ipcat, h, hdiff, wi, wgs, bg, wcs, bc):
        ipg = _mm(wi, jnp.concatenate((ipcat, h), axis=0))
        dcat = jnp.concatenate(hdiff, axis=0)
        g = jax.nn.sigmoid(ipg[:2 * _H] + _mm(wgs, dcat) + bg)
        r = g[:_H]
        u = g[_H:]
        rh = r * h
        rcat = jnp.concatenate((rh,) + diffuse(rh), axis=0)
        c = jnp.tanh(ipg[2 * _H:] + _mm(wcs, rcat) + bc)
        return u * h + (1.0 - u) * c

    def ipcat0_at(t, pxc):
        sl = slice(t * _D, (t + 1) * _D)
        sl4 = slice((t % 4) * _D, (t % 4 + 1) * _D)
        return jnp.concatenate(
            [jnp.concatenate([xs[i][sl] for i in range(_B)], axis=1)] +
            [jnp.concatenate([pxc[i][sl4, k * _N:(k + 1) * _N]
                              for i in range(_B)], axis=1)
             for k in range(4)], axis=0)

    def xchunk(t):
        csl = slice(t * _D, (t + 4) * _D)
        return [_mm(xs[i][csl], sup[i]) for i in range(_B)]

    pxc = xchunk(0)
    ipg = _mm(wi0[:, :_NUM_MAT * _D], ipcat0_at(0, pxc))
    u = jax.nn.sigmoid(ipg[_H:2 * _H] + bg0[_H:])
    c = jnp.tanh(ipg[2 * _H:] + bc0)
    h0 = (1.0 - u) * c
    h0diff = diffuse(h0)
    ipg1 = _mm(wi1[:, :_NUM_MAT * _H],
               jnp.concatenate((h0,) + h0diff, axis=0))
    u1 = jax.nn.sigmoid(ipg1[_H:2 * _H] + bg1[_H:])
    c1 = jnp.tanh(ipg1[2 * _H:] + bc1)
    h1 = (1.0 - u1) * c1

    for t in range(1, _T):
        if t % 4 == 0:
            pxc = xchunk(t)
        h0_new = gru(ipcat0_at(t, pxc), h0, h0diff,
                     wi0, wg0s, bg0, wc0s, bc0)
        both = jnp.concatenate([h0_new, h1], axis=0)
        bdiff = diffuse(both)
        h0diff = tuple(d[:_H] for d in bdiff)
        h1diff = tuple(d[_H:] for d in bdiff)
        ipcat1 = jnp.concatenate((h0_new,) + h0diff, axis=0)
        h1 = gru(ipcat1, h1, h1diff, wi1, wg1s, bg1, wc1s, bc1)
        h0 = h0_new

    lg = jnp.sum(jnp.maximum(h1, 0.0) * wfc_ref[...], axis=0,
                 keepdims=True) + bfc_ref[...]
    out_ref[...] = jnp.concatenate(
        [jnp.full((1, 1, _N), jnp.max(lg[:, i * _N:(i + 1) * _N]),
                  jnp.float32) for i in range(_B)], axis=0)


def _split_w(w, din):
    wr = w.reshape(_NUM_MAT, din + _H, -1)
    w_in = wr[:, :din, :].reshape(_NUM_MAT * din, -1)
    w_st = wr[:, din:, :].reshape(_NUM_MAT * _H, -1)
    return w_in, w_st


@jax.jit
def kernel(x, W_gate0, b_gate0, W_cand0, b_cand0, W_gate1, b_gate1,
           W_cand1, b_cand1, W_fc, b_fc):
    wg0i, wg0s = _split_w(W_gate0, _D)
    wc0i, wc0s = _split_w(W_cand0, _D)
    wg1i, wg1s = _split_w(W_gate1, _H)
    wc1i, wc1s = _split_w(W_cand1, _H)
    wi0 = jnp.concatenate([wg0i, wc0i], axis=1).T
    wi1 = jnp.concatenate([wg1i, wc1i], axis=1).T
    id0 = jnp.concatenate([wg0s[:_H], jnp.zeros((_H, _H), jnp.float32)],
                          axis=1).T
    id1 = jnp.concatenate([wg1s[:_H], jnp.zeros((_H, _H), jnp.float32)],
                          axis=1).T
    wi0 = jnp.concatenate([wi0, id0], axis=1)
    wi1 = jnp.concatenate([wi1, id1], axis=1)
    wg0s = wg0s[_H:]
    wg1s = wg1s[_H:]

    const = lambda b: (0, 0)
    wspec = lambda a: pl.BlockSpec(a.shape, const)
    operands = (x, wi0, wg0s.T, b_gate0.reshape(-1, 1),
                wc0s.T, b_cand0.reshape(-1, 1),
                wi1, wg1s.T, b_gate1.reshape(-1, 1),
                wc1s.T, b_cand1.reshape(-1, 1),
                W_fc, b_fc.reshape(1, 1))
    in_specs = [pl.BlockSpec((_B, _T, _N, _D), lambda b: (b, 0, 0, 0))]
    in_specs += [wspec(a) for a in operands[1:]]

    out = pl.pallas_call(
        _body,
        grid=(1,),
        in_specs=in_specs,
        out_specs=pl.BlockSpec((_B, 1, _N), lambda b: (b, 0, 0)),
        out_shape=jax.ShapeDtypeStruct((_B, 1, _N), jnp.float32),
        compiler_params=pltpu.CompilerParams(
            dimension_semantics=("arbitrary",)),
    )(*operands)
    return out[:, 0, 0]


# speedup vs baseline: 1.1682x; 1.1682x over previous
import jax
import jax.numpy as jnp
from jax.experimental import pallas as pl
from jax.experimental.pallas import tpu as pltpu

_B, _T, _N, _D, _H = 16, 12, 128, 64, 128
_NUM_MAT = 5
_M = _B * _N


def _mm(a, b):
    return jax.lax.dot_general(a, b, (((1,), (0,)), ((), ())),
                               preferred_element_type=jnp.float32)


def _mmr(a, b):
    return jax.lax.dot_general(a, b, (((0,), (0,)), ((), ())),
                               preferred_element_type=jnp.float32)


def _body(x_ref, wi0_ref, wg0s_ref, bg0_ref, wc0s_ref, bc0_ref,
          wi1_ref, wg1s_ref, bg1_ref, wc1s_ref, bc1_ref,
          wfc_ref, bfc_ref, out_ref):
    row = jax.lax.broadcasted_iota(jnp.int32, (_N, _N), 0)
    col = jax.lax.broadcasted_iota(jnp.int32, (_N, _N), 1)
    eye = jnp.where(row == col, 1.0, 0.0).astype(jnp.float32)
    sup = []
    xs = []
    for i in range(_B):
        xt_i = x_ref[i]
        xs.append(xt_i)
        mu = jnp.sum(xt_i, axis=0, keepdims=True) * (1.0 / (_T * _D))
        xm = xt_i - mu
        cov = _mmr(xm, xm)
        sq = jnp.sum(xm * xm, axis=0, keepdims=True)
        var = jnp.sqrt(jnp.maximum(sq, 1e-12))
        adj = jnp.abs(cov) / (jnp.transpose(var) * var)
        rs = jnp.sum(adj, axis=1, keepdims=True)
        cs = jnp.sum(adj, axis=0, keepdims=True)
        s0t = jnp.transpose(adj) / jnp.transpose(rs)
        s1t = adj / cs
        sup.append(jnp.concatenate(
            [s0t, _mm(s0t, s0t) * 2.0 - eye, s1t,
             _mm(s1t, s1t) * 2.0 - eye], axis=1))

    def diffuse(v):
        prods = [_mm(v[:, i * _N:(i + 1) * _N], sup[i]) for i in range(_B)]
        return tuple(
            jnp.concatenate([p[:, k * _N:(k + 1) * _N] for p in prods],
                            axis=1) for k in range(4))

    wi0 = wi0_ref[...]
    wg0s = wg0s_ref[...]
    bg0 = bg0_ref[...]
    wc0s = wc0s_ref[...]
    bc0 = bc0_ref[...]
    wi1 = wi1_ref[...]
    wg1s = wg1s_ref[...]
    bg1 = bg1_ref[...]
    wc1s = wc1s_ref[...]
    bc1 = bc1_ref[...]

    def gru(ipcat, h, hdiff, wi, wgs, bg, wcs, bc):
        ipg = _mm(wi, jnp.concatenate((ipcat, h), axis=0))
        dcat = jnp.concatenate(hdiff, axis=0)
        g = jax.nn.sigmoid(ipg[:2 * _H] + _mm(wgs, dcat) + bg)
        r = g[:_H]
        u = g[_H:]
        rh = r * h
        rcat = jnp.concatenate((rh,) + diffuse(rh), axis=0)
        c = jnp.tanh(ipg[2 * _H:] + _mm(wcs, rcat) + bc)
        return u * h + (1.0 - u) * c

    def ipcat0_at(t, pxc):
        sl = slice(t * _D, (t + 1) * _D)
        sl4 = slice((t % 4) * _D, (t % 4 + 1) * _D)
        return jnp.concatenate(
            [jnp.concatenate([xs[i][sl] for i in range(_B)], axis=1)] +
            [jnp.concatenate([pxc[i][sl4, k * _N:(k + 1) * _N]
                              for i in range(_B)], axis=1)
             for k in range(4)], axis=0)

    def xchunk(t):
        csl = slice(t * _D, (t + 4) * _D)
        return [_mm(xs[i][csl], sup[i]) for i in range(_B)]

    pxc = xchunk(0)
    ipg = _mm(wi0[:, :_NUM_MAT * _D], ipcat0_at(0, pxc))
    u = jax.nn.sigmoid(ipg[_H:2 * _H] + bg0[_H:])
    c = jnp.tanh(ipg[2 * _H:] + bc0)
    h0 = (1.0 - u) * c
    h0diff = diffuse(h0)
    ipg1 = _mm(wi1[:, :_NUM_MAT * _H],
               jnp.concatenate((h0,) + h0diff, axis=0))
    u1 = jax.nn.sigmoid(ipg1[_H:2 * _H] + bg1[_H:])
    c1 = jnp.tanh(ipg1[2 * _H:] + bc1)
    h1 = (1.0 - u1) * c1

    for t in range(1, _T):
        if t % 4 == 0:
            pxc = xchunk(t)
        h0_new = gru(ipcat0_at(t, pxc), h0, h0diff,
                     wi0, wg0s, bg0, wc0s, bc0)
        both = jnp.concatenate([h0_new, h1], axis=0)
        bdiff = diffuse(both)
        h0diff = tuple(d[:_H] for d in bdiff)
        h1diff = tuple(d[_H:] for d in bdiff)
        ipcat1 = jnp.concatenate((h0_new,) + h0diff, axis=0)
        h1 = gru(ipcat1, h1, h1diff, wi1, wg1s, bg1, wc1s, bc1)
        h0 = h0_new

    lg = jnp.sum(jnp.maximum(h1, 0.0) * wfc_ref[...], axis=0,
                 keepdims=True) + bfc_ref[...]
    out_ref[...] = jnp.concatenate(
        [jnp.full((1, 1, _N), jnp.max(lg[:, i * _N:(i + 1) * _N]),
                  jnp.float32) for i in range(_B)], axis=0)


def _split_w(w, din):
    wr = w.reshape(_NUM_MAT, din + _H, -1)
    w_in = wr[:, :din, :].reshape(_NUM_MAT * din, -1)
    w_st = wr[:, din:, :].reshape(_NUM_MAT * _H, -1)
    return w_in, w_st


@jax.jit
def kernel(x, W_gate0, b_gate0, W_cand0, b_cand0, W_gate1, b_gate1,
           W_cand1, b_cand1, W_fc, b_fc):
    wg0i, wg0s = _split_w(W_gate0, _D)
    wc0i, wc0s = _split_w(W_cand0, _D)
    wg1i, wg1s = _split_w(W_gate1, _H)
    wc1i, wc1s = _split_w(W_cand1, _H)
    wi0 = jnp.concatenate([wg0i, wc0i], axis=1).T
    wi1 = jnp.concatenate([wg1i, wc1i], axis=1).T
    id0 = jnp.concatenate([wg0s[:_H], jnp.zeros((_H, _H), jnp.float32)],
                          axis=1).T
    id1 = jnp.concatenate([wg1s[:_H], jnp.zeros((_H, _H), jnp.float32)],
                          axis=1).T
    wi0 = jnp.concatenate([wi0, id0], axis=1)
    wi1 = jnp.concatenate([wi1, id1], axis=1)
    wg0s = wg0s[_H:]
    wg1s = wg1s[_H:]

    xp = jnp.transpose(x, (0, 1, 3, 2)).reshape(_B, _T * _D, _N)

    const = lambda b: (0, 0)
    wspec = lambda a: pl.BlockSpec(a.shape, const)
    operands = (xp, wi0, wg0s.T, b_gate0.reshape(-1, 1),
                wc0s.T, b_cand0.reshape(-1, 1),
                wi1, wg1s.T, b_gate1.reshape(-1, 1),
                wc1s.T, b_cand1.reshape(-1, 1),
                W_fc, b_fc.reshape(1, 1))
    in_specs = [pl.BlockSpec((_B, _T * _D, _N), lambda b: (b, 0, 0))]
    in_specs += [wspec(a) for a in operands[1:]]

    out = pl.pallas_call(
        _body,
        grid=(1,),
        in_specs=in_specs,
        out_specs=pl.BlockSpec((_B, 1, _N), lambda b: (b, 0, 0)),
        out_shape=jax.ShapeDtypeStruct((_B, 1, _N), jnp.float32),
        compiler_params=pltpu.CompilerParams(
            dimension_semantics=("arbitrary",)),
    )(*operands)
    return out[:, 0, 0]
